# baseline (speedup 1.0000x reference)
import jax
import jax.numpy as jnp
from jax import lax
from jax.experimental import pallas as pl
from jax.experimental.pallas import tpu as pltpu

NUM_PERM = 8
C = 8
N = 1024
D = 768

_ROWS = NUM_PERM * C * N
_NDMA = 8
_STRIPE = _ROWS // _NDMA


def _dma_body(in_hbm, out_hbm, argsort_ref, perm_ref, sem):
    copies = [
        pltpu.make_async_copy(
            in_hbm.at[pl.ds(i * _STRIPE, _STRIPE)],
            out_hbm.at[pl.ds(i * _STRIPE, _STRIPE)],
            sem,
        )
        for i in range(_NDMA)
    ]
    for c in copies:
        c.start()
    argsort_ref[...] = lax.broadcasted_iota(jnp.int32, (NUM_PERM, N), 1)
    perm_ref[...] = lax.broadcasted_iota(jnp.int32, (1, NUM_PERM), 1)
    for c in copies:
        c.wait()


_copy = pl.pallas_call(
    _dma_body,
    in_specs=[pl.BlockSpec(memory_space=pltpu.MemorySpace.HBM)],
    out_specs=[
        pl.BlockSpec(memory_space=pltpu.MemorySpace.HBM),
        pl.BlockSpec(memory_space=pltpu.MemorySpace.VMEM),
        pl.BlockSpec(memory_space=pltpu.MemorySpace.VMEM),
    ],
    out_shape=[
        jax.ShapeDtypeStruct((_ROWS, D), jnp.float32),
        jax.ShapeDtypeStruct((NUM_PERM, N), jnp.int32),
        jax.ShapeDtypeStruct((1, NUM_PERM), jnp.int32),
    ],
    scratch_shapes=[pltpu.SemaphoreType.DMA],
)


def kernel(patches):
    aug, argsort, perm2d = _copy(patches.reshape(_ROWS, D))
    return (aug.reshape(NUM_PERM, C, N, D), argsort, perm2d.reshape(NUM_PERM))


# speedup vs baseline: 43.0295x; 43.0295x over previous
import jax
import jax.numpy as jnp
from jax import lax
from jax.experimental import pallas as pl
from jax.experimental.pallas import tpu as pltpu
from jax.experimental.pallas import tpu_sc as plsc

NUM_PERM = 8
C = 8
N = 1024
D = 768

_ROWS = NUM_PERM * C * N
_BLOCK_ROWS = 4096

_NC = 2
_NS = 16
_NW = _NC * _NS
_ACHUNK = (NUM_PERM * N) // _NW


def _copy_body(in_ref, out_ref):
    out_ref[...] = in_ref[...]


_tc_copy = pl.pallas_call(
    _copy_body,
    grid=(_ROWS // _BLOCK_ROWS,),
    in_specs=[pl.BlockSpec((_BLOCK_ROWS, D), lambda i: (i, 0))],
    out_specs=pl.BlockSpec((_BLOCK_ROWS, D), lambda i: (i, 0)),
    out_shape=jax.ShapeDtypeStruct((_ROWS, D), jnp.float32),
)


def _sc_idx_body(argsort_hbm, perm_hbm, asort_v, perm_v):
    cid = lax.axis_index("c")
    sid = lax.axis_index("s")
    wid = sid * _NC + cid

    abase = wid * _ACHUNK
    row_off = lax.rem(abase, N)
    for v in range(_ACHUNK // 16):
        asort_v[pl.ds(v * 16, 16)] = lax.iota(jnp.int32, 16) + (row_off + v * 16)
    pltpu.sync_copy(asort_v, argsort_hbm.at[pl.ds(abase, _ACHUNK)])

    @pl.when(wid == 0)
    def _():
        perm_v[...] = lax.iota(jnp.int32, 16)
        pltpu.sync_copy(perm_v, perm_hbm)


_sc_idx = pl.kernel(
    _sc_idx_body,
    out_type=(
        jax.ShapeDtypeStruct((NUM_PERM * N,), jnp.int32),
        jax.ShapeDtypeStruct((16,), jnp.int32),
    ),
    mesh=plsc.VectorSubcoreMesh(core_axis_name="c", subcore_axis_name="s"),
    scratch_types=[
        pltpu.VMEM((_ACHUNK,), jnp.int32),
        pltpu.VMEM((16,), jnp.int32),
    ],
)


def kernel(patches):
    argsort_flat, perm16 = _sc_idx()
    aug = _tc_copy(patches.reshape(_ROWS, D))
    return (
        aug.reshape(NUM_PERM, C, N, D),
        argsort_flat.reshape(NUM_PERM, N),
        perm16[:NUM_PERM],
    )


# speedup vs baseline: 48.9635x; 1.1379x over previous
import jax
import jax.numpy as jnp
from jax import lax
from jax.experimental import pallas as pl
from jax.experimental.pallas import tpu as pltpu

NUM_PERM = 8
C = 8
N = 1024
D = 768

_ROWS = NUM_PERM * C * N
_BLOCK_ROWS = 4096


def _copy_body(in_ref, out_ref, argsort_ref, perm_ref):
    out_ref[...] = in_ref[...]
    argsort_ref[...] = lax.broadcasted_iota(jnp.int32, (NUM_PERM, N), 1)
    perm_ref[...] = lax.broadcasted_iota(jnp.int32, (1, NUM_PERM), 1)


_copy = pl.pallas_call(
    _copy_body,
    grid=(_ROWS // _BLOCK_ROWS,),
    in_specs=[pl.BlockSpec((_BLOCK_ROWS, D), lambda i: (i, 0))],
    out_specs=[
        pl.BlockSpec((_BLOCK_ROWS, D), lambda i: (i, 0)),
        pl.BlockSpec((NUM_PERM, N), lambda i: (0, 0)),
        pl.BlockSpec((1, NUM_PERM), lambda i: (0, 0)),
    ],
    out_shape=[
        jax.ShapeDtypeStruct((_ROWS, D), jnp.float32),
        jax.ShapeDtypeStruct((NUM_PERM, N), jnp.int32),
        jax.ShapeDtypeStruct((1, NUM_PERM), jnp.int32),
    ],
    compiler_params=pltpu.CompilerParams(dimension_semantics=("parallel",)),
)


def kernel(patches):
    aug, argsort, perm2d = _copy(patches.reshape(_ROWS, D))
    return (aug.reshape(NUM_PERM, C, N, D), argsort, perm2d.reshape(NUM_PERM))
